# Initial kernel scaffold; baseline (speedup 1.0000x reference)
import jax
import jax.numpy as jnp
from jax.experimental import pallas as pl


def kernel(h_src, h_dst, edge_index, Wq_w, Wq_b, Wk_w, Wk_b, Wv_w, Wv_b, proj_w, proj_b, ln_g, ln_b):
    raise NotImplementedError("write your pallas kernel here")


# speedup vs baseline: 1.0008x; 1.0008x over previous
import functools
import numpy as np
import jax
import jax.numpy as jnp
from jax import lax
from jax.experimental import pallas as pl
from jax.experimental.pallas import tpu as pltpu

N = 10000
E = 160000
DIM = 256
HEADS = 8
DK = DIM // HEADS

ROW_BLK = 1000


def _qkv_body(hd_ref, hs_ref, wq_ref, wk_ref, wv_ref, bq_ref, bk_ref, bv_ref,
              q_ref, k_ref, v_ref):
    dn = (((1,), (1,)), ((), ()))
    q_ref[...] = lax.dot_general(hd_ref[...], wq_ref[...], dn,
                                 preferred_element_type=jnp.float32) + bq_ref[...][None, :]
    k_ref[...] = lax.dot_general(hs_ref[...], wk_ref[...], dn,
                                 preferred_element_type=jnp.float32) + bk_ref[...][None, :]
    v_ref[...] = lax.dot_general(hs_ref[...], wv_ref[...], dn,
                                 preferred_element_type=jnp.float32) + bv_ref[...][None, :]


def _qkv(h_dst, h_src, Wq_w, Wk_w, Wv_w, Wq_b, Wk_b, Wv_b):
    grid = (N // ROW_BLK,)
    row_spec = pl.BlockSpec((ROW_BLK, DIM), lambda i: (i, 0))
    w_spec = pl.BlockSpec((DIM, DIM), lambda i: (0, 0))
    b_spec = pl.BlockSpec((DIM,), lambda i: (0,))
    return pl.pallas_call(
        _qkv_body,
        grid=grid,
        in_specs=[row_spec, row_spec, w_spec, w_spec, w_spec, b_spec, b_spec, b_spec],
        out_specs=[row_spec, row_spec, row_spec],
        out_shape=[jax.ShapeDtypeStruct((N, DIM), jnp.float32)] * 3,
    )(h_dst, h_src, Wq_w, Wk_w, Wv_w, Wq_b, Wk_b, Wv_b)


def _proj_ln_body(agg_ref, hd_ref, pw_ref, pb_ref, g_ref, b_ref, y_ref):
    dn = (((1,), (1,)), ((), ()))
    out = lax.dot_general(agg_ref[...], pw_ref[...], dn,
                          preferred_element_type=jnp.float32) + pb_ref[...][None, :]
    res = hd_ref[...] + out
    mu = jnp.mean(res, axis=-1, keepdims=True)
    var = jnp.mean((res - mu) * (res - mu), axis=-1, keepdims=True)
    y_ref[...] = (res - mu) * lax.rsqrt(var + 1e-5) * g_ref[...][None, :] + b_ref[...][None, :]


def _proj_ln(agg, h_dst, proj_w, proj_b, ln_g, ln_b):
    grid = (N // ROW_BLK,)
    row_spec = pl.BlockSpec((ROW_BLK, DIM), lambda i: (i, 0))
    w_spec = pl.BlockSpec((DIM, DIM), lambda i: (0, 0))
    b_spec = pl.BlockSpec((DIM,), lambda i: (0,))
    return pl.pallas_call(
        _proj_ln_body,
        grid=grid,
        in_specs=[row_spec, row_spec, w_spec, b_spec, b_spec, b_spec],
        out_specs=row_spec,
        out_shape=jax.ShapeDtypeStruct((N, DIM), jnp.float32),
    )(agg, h_dst, proj_w, proj_b, ln_g, ln_b)


def kernel(h_src, h_dst, edge_index, Wq_w, Wq_b, Wk_w, Wk_b, Wv_w, Wv_b,
           proj_w, proj_b, ln_g, ln_b):
    src = edge_index[0]
    dst = edge_index[1]
    q, k, v = _qkv(h_dst, h_src, Wq_w, Wk_w, Wv_w, Wq_b, Wk_b, Wv_b)

    qh = q.reshape(N, HEADS, DK)
    kh = k.reshape(N, HEADS, DK)
    vh = v.reshape(N, HEADS, DK)
    score = jnp.sum(kh[src] * qh[dst], axis=-1) / np.sqrt(DK)
    smax = jax.ops.segment_max(score, dst, num_segments=N)
    ex = jnp.exp(score - smax[dst])
    denom = jax.ops.segment_sum(ex, dst, num_segments=N)
    attn = ex / (denom[dst] + 1e-9)
    msg = vh[src] * attn[:, :, None]
    agg = jax.ops.segment_sum(msg, dst, num_segments=N).reshape(N, DIM)

    return _proj_ln(agg, h_dst, proj_w, proj_b, ln_g, ln_b)


# speedup vs baseline: 5.2316x; 5.2272x over previous
import functools
import numpy as np
import jax
import jax.numpy as jnp
from jax import lax
from jax.experimental import pallas as pl
from jax.experimental.pallas import tpu as pltpu
from jax.experimental.pallas import tpu_sc as plsc

N = 10000
E = 160000
DIM = 256
HEADS = 8
DK = DIM // HEADS

NC = 2
NS = 16
NW = NC * NS
L = 16

B = 128
EPAD = 163840
EPW = EPAD // NW
BLKA = EPW // B
EPT = EPAD // NS
BLKB = EPT // B
HH = HEADS // NC
CH = DIM // NC
HQ = 2
CQ = HQ * DK
DEN_W = 8
RPT = N // NS

ROW_BLK = 1000

_mesh = plsc.VectorSubcoreMesh(core_axis_name="c", subcore_axis_name="s")
_sc_params = pltpu.CompilerParams(use_tc_tiling_on_sc=False,
                                  needs_layout_passes=False)


def _qkv_body(hd_ref, hs_ref, wq_ref, wk_ref, wv_ref, bq_ref, bk_ref, bv_ref,
              q_ref, k_ref, v2_ref):
    dn = (((1,), (1,)), ((), ()))
    q = lax.dot_general(hd_ref[...], wq_ref[...], dn,
                        preferred_element_type=jnp.float32) + bq_ref[...][None, :]
    q_ref[...] = q * (1.0 / np.sqrt(DK))
    k_ref[...] = lax.dot_general(hs_ref[...], wk_ref[...], dn,
                                 preferred_element_type=jnp.float32) + bk_ref[...][None, :]
    v = lax.dot_general(hs_ref[...], wv_ref[...], dn,
                        preferred_element_type=jnp.float32) + bv_ref[...][None, :]
    for qq in range(4):
        v2_ref[qq] = v[:, qq * CQ:(qq + 1) * CQ]


def _qkv(h_dst, h_src, Wq_w, Wk_w, Wv_w, Wq_b, Wk_b, Wv_b):
    grid = (N // ROW_BLK,)
    row_spec = pl.BlockSpec((ROW_BLK, DIM), lambda i: (i, 0))
    w_spec = pl.BlockSpec((DIM, DIM), lambda i: (0, 0))
    b_spec = pl.BlockSpec((DIM,), lambda i: (0,))
    v2_spec = pl.BlockSpec((4, ROW_BLK, CQ), lambda i: (0, i, 0))
    return pl.pallas_call(
        _qkv_body,
        grid=grid,
        in_specs=[row_spec, row_spec, w_spec, w_spec, w_spec, b_spec, b_spec, b_spec],
        out_specs=[row_spec, row_spec, v2_spec],
        out_shape=[jax.ShapeDtypeStruct((N, DIM), jnp.float32),
                   jax.ShapeDtypeStruct((N, DIM), jnp.float32),
                   jax.ShapeDtypeStruct((4, N, CQ), jnp.float32)],
    )(h_dst, h_src, Wq_w, Wk_w, Wv_w, Wq_b, Wk_b, Wv_b)


def _score_body(k_hbm, q_hbm, src_hbm, dstq_hbm,
                score_hbm, wmax_hbm,
                src_v, dstq_v, krows, qrows, score_v, wbuf, semk, semq):
    w = lax.axis_index("s") * NC + lax.axis_index("c")
    pltpu.sync_copy(src_hbm.at[pl.ds(w * BLKA, BLKA)], src_v)
    pltpu.sync_copy(dstq_hbm.at[pl.ds(w * BLKA, BLKA)], dstq_v)

    iota = lax.iota(jnp.int32, L)

    def block_body(j, m):
        cpk = pltpu.async_copy(k_hbm.at[src_v.at[j]], krows, semk)
        cpq = pltpu.async_copy(q_hbm.at[dstq_v.at[j]], qrows, semq)
        cpk.wait()
        cpq.wait()
        for g in range(B // L):
            rows = iota + (g * L)

            def d_body(d, accs):
                out = []
                for h in range(HEADS):
                    col = jnp.broadcast_to(h * DK + d, (L,)).astype(jnp.int32)
                    kv = plsc.load_gather(krows, [rows, col])
                    qv = plsc.load_gather(qrows, [rows, col])
                    out.append(accs[h] + kv * qv)
                return tuple(out)

            accs = lax.fori_loop(
                0, DK, d_body,
                tuple(jnp.zeros((L,), jnp.float32) for _ in range(HEADS)))
            off = j * B + g * L
            for h in range(HEADS):
                score_v[h, pl.ds(off, L)] = accs[h]
                m = jnp.maximum(m, accs[h])
        return m

    m = lax.fori_loop(0, BLKA, block_body, jnp.full((L,), -3e38, jnp.float32))
    wbuf[...] = m
    pltpu.sync_copy(wbuf, wmax_hbm.at[w])
    for h in range(HEADS):
        pltpu.sync_copy(score_v.at[h], score_hbm.at[h, pl.ds(w * EPW, EPW)])


_score_call = functools.partial(
    pl.kernel,
    out_type=[jax.ShapeDtypeStruct((HEADS, EPAD), jnp.float32),
              jax.ShapeDtypeStruct((NW, L), jnp.float32)],
    mesh=_mesh,
    compiler_params=_sc_params,
    scratch_types=[
        pltpu.VMEM((BLKA, B), jnp.int32),
        pltpu.VMEM((BLKA, B), jnp.int32),
        pltpu.VMEM((B, DIM), jnp.float32),
        pltpu.VMEM((B, DIM), jnp.float32),
        pltpu.VMEM((HEADS, EPW), jnp.float32),
        pltpu.VMEM((L,), jnp.float32),
        pltpu.SemaphoreType.DMA,
        pltpu.SemaphoreType.DMA,
    ],
)(_score_body)


def _make_agg_body(hg):
    def _agg_body(score_hbm, wmax_hbm, v2_hbm, src_hbm, dsts_hbm,
                  zacc_hbm, zden_hbm,
                  out_hbm,
                  src_v, dsts_v, ex_hm, exb, vrows, obuf, wmaxv, denb,
                  acc_sp, den_sp, semv):
        c = lax.axis_index("c")
        t = lax.axis_index("s")
        qq = hg * NC + c

        pltpu.sync_copy(wmax_hbm, wmaxv)
        m = wmaxv[0, :]
        for i in range(1, NW):
            m = jnp.maximum(m, wmaxv[i, :])
        gmax = jnp.max(m)

        rb = t * RPT
        pltpu.sync_copy(zacc_hbm.at[pl.ds(rb, RPT)], acc_sp.at[pl.ds(rb, RPT)])
        pltpu.sync_copy(zden_hbm.at[pl.ds(rb, RPT)], den_sp.at[pl.ds(rb, RPT)])

        pltpu.sync_copy(src_hbm.at[pl.ds(t * BLKB, BLKB)], src_v)
        pltpu.sync_copy(dsts_hbm.at[pl.ds(t * BLKB, BLKB)], dsts_v)

        iota = lax.iota(jnp.int32, L)
        ebase = t * EPT

        for g in range(B // L):
            rows = iota + g * L
            for h in range(DEN_W):
                plsc.store_scatter(exb, [rows, jnp.full((L,), h, jnp.int32)],
                                   jnp.zeros((L,), jnp.float32))

        for h in range(HQ):
            pltpu.sync_copy(score_hbm.at[qq * HQ + h, pl.ds(ebase, EPT)],
                            ex_hm.at[h])

            def ex_body(i, _, h=h):
                sv = ex_hm[h, pl.ds(i * L, L)]
                ev = jnp.exp(sv - gmax)
                gid = iota + (ebase + i * L)
                ev = jnp.where(gid < E, ev, 0.0)
                ex_hm[h, pl.ds(i * L, L)] = ev
                return 0

            lax.fori_loop(0, EPT // L, ex_body, 0)

        plsc.subcore_barrier()

        def den_body(j, _):
            for g in range(B // L):
                rows = iota + g * L
                for h in range(HQ):
                    ev = ex_hm[h, pl.ds(j * B + g * L, L)]
                    plsc.store_scatter(
                        exb, [rows, jnp.full((L,), h, jnp.int32)], ev)
            pltpu.sync_copy(exb, den_sp.at[dsts_v.at[j]], add=True)
            return 0

        lax.fori_loop(0, BLKB, den_body, 0)

        def blk_body(j, _):
            pltpu.async_copy(v2_hbm.at[qq].at[src_v.at[j]], vrows, semv).wait()
            for g in range(B // L):
                rows = iota + g * L
                exvs = [ex_hm[h, pl.ds(j * B + g * L, L)] for h in range(HQ)]

                def d2_body(d, _, rows=rows, exvs=exvs):
                    for h in range(HQ):
                        col = jnp.broadcast_to(h * DK + d, (L,)).astype(jnp.int32)
                        vv = plsc.load_gather(vrows, [rows, col])
                        plsc.store_scatter(vrows, [rows, col], vv * exvs[h])
                    return 0

                lax.fori_loop(0, DK, d2_body, 0)
            pltpu.sync_copy(vrows, acc_sp.at[dsts_v.at[j]], add=True)
            return 0

        lax.fori_loop(0, BLKB, blk_body, 0)

        plsc.subcore_barrier()

        for off, sz in ((0, 128), (128, 128), (256, 128), (384, 128), (512, 113)):
            rs = rb + off
            pltpu.sync_copy(acc_sp.at[pl.ds(rs, sz)], vrows.at[pl.ds(0, sz)])
            pltpu.sync_copy(den_sp.at[pl.ds(rs, sz)], denb.at[pl.ds(0, sz)])
            for g in range(B // L):
                rows = iota + g * L
                rvs = []
                for h in range(HQ):
                    dv = plsc.load_gather(
                        denb, [rows, jnp.full((L,), h, jnp.int32)])
                    rvs.append(1.0 / (dv + 1e-9))

                def d3_body(d, _, rows=rows, rvs=rvs):
                    for h in range(HQ):
                        col = jnp.broadcast_to(h * DK + d, (L,)).astype(jnp.int32)
                        av = plsc.load_gather(vrows, [rows, col])
                        plsc.store_scatter(obuf, [rows, col], av * rvs[h])
                    return 0

                lax.fori_loop(0, DK, d3_body, 0)
            pltpu.sync_copy(obuf.at[pl.ds(0, sz)], out_hbm.at[c, pl.ds(rs, sz)])

    return _agg_body


def _make_agg_call(hg):
    return functools.partial(
        pl.kernel,
        out_type=jax.ShapeDtypeStruct((NC, N, CQ), jnp.float32),
        mesh=_mesh,
        compiler_params=_sc_params,
        scratch_types=[
            pltpu.VMEM((BLKB, B), jnp.int32),
            pltpu.VMEM((BLKB, B), jnp.int32),
            pltpu.VMEM((HQ, EPT), jnp.float32),
            pltpu.VMEM((B, DEN_W), jnp.float32),
            pltpu.VMEM((B, CQ), jnp.float32),
            pltpu.VMEM((B, CQ), jnp.float32),
            pltpu.VMEM((NW, L), jnp.float32),
            pltpu.VMEM((B, DEN_W), jnp.float32),
            pltpu.VMEM_SHARED((N, CQ), jnp.float32),
            pltpu.VMEM_SHARED((N, DEN_W), jnp.float32),
            pltpu.SemaphoreType.DMA,
        ],
    )(_make_agg_body(hg))


_agg_call_0 = _make_agg_call(0)
_agg_call_1 = _make_agg_call(1)


def _proj_ln_body(agg_ref, hd_ref, pw_ref, pb_ref, g_ref, b_ref, y_ref):
    dn = (((1,), (1,)), ((), ()))
    out = pb_ref[...][None, :]
    for qq in range(4):
        out = out + lax.dot_general(agg_ref[qq], pw_ref[:, qq * CQ:(qq + 1) * CQ],
                                    dn, preferred_element_type=jnp.float32)
    res = hd_ref[...] + out
    mu = jnp.mean(res, axis=-1, keepdims=True)
    var = jnp.mean((res - mu) * (res - mu), axis=-1, keepdims=True)
    y_ref[...] = (res - mu) * lax.rsqrt(var + 1e-5) * g_ref[...][None, :] + b_ref[...][None, :]


def _proj_ln(agg4, h_dst, proj_w, proj_b, ln_g, ln_b):
    grid = (N // ROW_BLK,)
    row_spec = pl.BlockSpec((ROW_BLK, DIM), lambda i: (i, 0))
    agg_spec = pl.BlockSpec((4, ROW_BLK, CQ), lambda i: (0, i, 0))
    w_spec = pl.BlockSpec((DIM, DIM), lambda i: (0, 0))
    b_spec = pl.BlockSpec((DIM,), lambda i: (0,))
    return pl.pallas_call(
        _proj_ln_body,
        grid=grid,
        in_specs=[agg_spec, row_spec, w_spec, b_spec, b_spec, b_spec],
        out_specs=row_spec,
        out_shape=jax.ShapeDtypeStruct((N, DIM), jnp.float32),
    )(agg4, h_dst, proj_w, proj_b, ln_g, ln_b)


def kernel(h_src, h_dst, edge_index, Wq_w, Wq_b, Wk_w, Wk_b, Wv_w, Wv_b,
           proj_w, proj_b, ln_g, ln_b):
    src = edge_index[0]
    dst = edge_index[1]
    padz = jnp.zeros((EPAD - E,), jnp.int32)
    src_p = jnp.concatenate([src, padz]).reshape(EPAD // B, B)
    dst_p = jnp.concatenate([dst, padz]).reshape(EPAD // B, B)

    q, k, v2 = _qkv(h_dst, h_src, Wq_w, Wk_w, Wv_w, Wq_b, Wk_b, Wv_b)

    score, wmax = _score_call(k, q, src_p, dst_p)

    zacc = jnp.zeros((N, CQ), jnp.float32)
    zden = jnp.zeros((N, DEN_W), jnp.float32)
    agg_a = _agg_call_0(score, wmax, v2, src_p, dst_p, zacc, zden)
    agg_b = _agg_call_1(score, wmax, v2, src_p, dst_p, zacc, zden)
    agg4 = jnp.concatenate([agg_a, agg_b], axis=0)

    return _proj_ln(agg4, h_dst, proj_w, proj_b, ln_g, ln_b)


# speedup vs baseline: 5.2892x; 1.0110x over previous
import functools
import numpy as np
import jax
import jax.numpy as jnp
from jax import lax
from jax.experimental import pallas as pl
from jax.experimental.pallas import tpu as pltpu
from jax.experimental.pallas import tpu_sc as plsc

N = 10000
E = 160000
DIM = 256
HEADS = 8
DK = DIM // HEADS

NC = 2
NS = 16
NW = NC * NS
L = 16

B = 128
EPAD = 163840
EPW = EPAD // NW
BLKA = EPW // B
EPT = EPAD // NS
BLKB = EPT // B
HH = HEADS // NC
CH = DIM // NC
HQ = 2
CQ = HQ * DK
DEN_W = 8
RPT = N // NS

ROW_BLK = 1000

_mesh = plsc.VectorSubcoreMesh(core_axis_name="c", subcore_axis_name="s")
_sc_params = pltpu.CompilerParams(use_tc_tiling_on_sc=False,
                                  needs_layout_passes=False)


def _qkv_body(hd_ref, hs_ref, wq_ref, wk_ref, wv_ref, bq_ref, bk_ref, bv_ref,
              q_ref, k_ref, v2_ref):
    dn = (((1,), (1,)), ((), ()))
    q = lax.dot_general(hd_ref[...], wq_ref[...], dn,
                        preferred_element_type=jnp.float32) + bq_ref[...][None, :]
    q_ref[...] = q * (1.0 / np.sqrt(DK))
    k_ref[...] = lax.dot_general(hs_ref[...], wk_ref[...], dn,
                                 preferred_element_type=jnp.float32) + bk_ref[...][None, :]
    v = lax.dot_general(hs_ref[...], wv_ref[...], dn,
                        preferred_element_type=jnp.float32) + bv_ref[...][None, :]
    for qq in range(4):
        v2_ref[qq] = v[:, qq * CQ:(qq + 1) * CQ]


def _qkv(h_dst, h_src, Wq_w, Wk_w, Wv_w, Wq_b, Wk_b, Wv_b):
    grid = (N // ROW_BLK,)
    row_spec = pl.BlockSpec((ROW_BLK, DIM), lambda i: (i, 0))
    w_spec = pl.BlockSpec((DIM, DIM), lambda i: (0, 0))
    b_spec = pl.BlockSpec((DIM,), lambda i: (0,))
    v2_spec = pl.BlockSpec((4, ROW_BLK, CQ), lambda i: (0, i, 0))
    return pl.pallas_call(
        _qkv_body,
        grid=grid,
        in_specs=[row_spec, row_spec, w_spec, w_spec, w_spec, b_spec, b_spec, b_spec],
        out_specs=[row_spec, row_spec, v2_spec],
        out_shape=[jax.ShapeDtypeStruct((N, DIM), jnp.float32),
                   jax.ShapeDtypeStruct((N, DIM), jnp.float32),
                   jax.ShapeDtypeStruct((4, N, CQ), jnp.float32)],
    )(h_dst, h_src, Wq_w, Wk_w, Wv_w, Wq_b, Wk_b, Wv_b)


def _score_body(k_hbm, q_hbm, src_hbm, dstq_hbm,
                score_hbm, wmax_hbm,
                src_v, dstq_v, krows, qrows, score_v, wbuf, semk, semq):
    w = lax.axis_index("s") * NC + lax.axis_index("c")
    pltpu.sync_copy(src_hbm.at[pl.ds(w * BLKA, BLKA)], src_v)
    pltpu.sync_copy(dstq_hbm.at[pl.ds(w * BLKA, BLKA)], dstq_v)

    iota = lax.iota(jnp.int32, L)

    def block_body(j, m):
        cpk = pltpu.async_copy(k_hbm.at[src_v.at[j]], krows, semk)
        cpq = pltpu.async_copy(q_hbm.at[dstq_v.at[j]], qrows, semq)
        cpk.wait()
        cpq.wait()

        def group_body(g, m):
            rows = iota + g * L

            def d_body(d, accs):
                out = []
                for h in range(HEADS):
                    col = jnp.broadcast_to(h * DK + d, (L,)).astype(jnp.int32)
                    kv = plsc.load_gather(krows, [rows, col])
                    qv = plsc.load_gather(qrows, [rows, col])
                    out.append(accs[h] + kv * qv)
                return tuple(out)

            accs = lax.fori_loop(
                0, DK, d_body,
                tuple(jnp.zeros((L,), jnp.float32) for _ in range(HEADS)),
                unroll=8)
            off = j * B + g * L
            for h in range(HEADS):
                score_v[h, pl.ds(off, L)] = accs[h]
                m = jnp.maximum(m, accs[h])
            return m

        return lax.fori_loop(0, B // L, group_body, m)

    m = lax.fori_loop(0, BLKA, block_body, jnp.full((L,), -3e38, jnp.float32))
    wbuf[...] = m
    pltpu.sync_copy(wbuf, wmax_hbm.at[w])
    for h in range(HEADS):
        pltpu.sync_copy(score_v.at[h], score_hbm.at[h, pl.ds(w * EPW, EPW)])


_score_call = functools.partial(
    pl.kernel,
    out_type=[jax.ShapeDtypeStruct((HEADS, EPAD), jnp.float32),
              jax.ShapeDtypeStruct((NW, L), jnp.float32)],
    mesh=_mesh,
    compiler_params=_sc_params,
    scratch_types=[
        pltpu.VMEM((BLKA, B), jnp.int32),
        pltpu.VMEM((BLKA, B), jnp.int32),
        pltpu.VMEM((B, DIM), jnp.float32),
        pltpu.VMEM((B, DIM), jnp.float32),
        pltpu.VMEM((HEADS, EPW), jnp.float32),
        pltpu.VMEM((L,), jnp.float32),
        pltpu.SemaphoreType.DMA,
        pltpu.SemaphoreType.DMA,
    ],
)(_score_body)


def _make_agg_body(hg):
    def _agg_body(score_hbm, wmax_hbm, v2_hbm, src_hbm, dsts_hbm,
                  zacc_hbm, zden_hbm,
                  out_hbm,
                  src_v, dsts_v, ex_hm, exb, vrows, msgb, obuf, wmaxv, denb,
                  acc_sp, den_sp, semv):
        c = lax.axis_index("c")
        t = lax.axis_index("s")
        qq = hg * NC + c

        pltpu.sync_copy(wmax_hbm, wmaxv)
        m = wmaxv[0, :]
        for i in range(1, NW):
            m = jnp.maximum(m, wmaxv[i, :])
        gmax = jnp.max(m)

        rb = t * RPT
        pltpu.sync_copy(zacc_hbm.at[pl.ds(rb, RPT)], acc_sp.at[pl.ds(rb, RPT)])
        pltpu.sync_copy(zden_hbm.at[pl.ds(rb, RPT)], den_sp.at[pl.ds(rb, RPT)])

        pltpu.sync_copy(src_hbm.at[pl.ds(t * BLKB, BLKB)], src_v)
        pltpu.sync_copy(dsts_hbm.at[pl.ds(t * BLKB, BLKB)], dsts_v)

        iota = lax.iota(jnp.int32, L)
        ebase = t * EPT

        for g in range(B // L):
            rows = iota + g * L
            for h in range(DEN_W):
                plsc.store_scatter(exb, [rows, jnp.full((L,), h, jnp.int32)],
                                   jnp.zeros((L,), jnp.float32))

        for h in range(HQ):
            pltpu.sync_copy(score_hbm.at[qq * HQ + h, pl.ds(ebase, EPT)],
                            ex_hm.at[h])

            def ex_body(i, _, h=h):
                sv = ex_hm[h, pl.ds(i * L, L)]
                ev = jnp.exp(sv - gmax)
                gid = iota + (ebase + i * L)
                ev = jnp.where(gid < E, ev, 0.0)
                ex_hm[h, pl.ds(i * L, L)] = ev
                return 0

            lax.fori_loop(0, EPT // L, ex_body, 0, unroll=8)

        plsc.subcore_barrier()

        def den_body(j, _):
            def deng_body(g, _):
                rows = iota + g * L
                for h in range(HQ):
                    ev = ex_hm[h, pl.ds(j * B + g * L, L)]
                    plsc.store_scatter(
                        exb, [rows, jnp.full((L,), h, jnp.int32)], ev)
                return 0

            lax.fori_loop(0, B // L, deng_body, 0, unroll=4)
            pltpu.sync_copy(exb, den_sp.at[dsts_v.at[j]], add=True)
            return 0

        lax.fori_loop(0, BLKB, den_body, 0)

        def blk_body(j, _):
            pltpu.async_copy(v2_hbm.at[qq].at[src_v.at[j]], vrows, semv).wait()

            def g2_body(g, _):
                rows = iota + g * L
                exvs = [ex_hm[h, pl.ds(j * B + g * L, L)] for h in range(HQ)]
                for d in range(DK):
                    for h in range(HQ):
                        col = jnp.broadcast_to(h * DK + d, (L,)).astype(jnp.int32)
                        vv = plsc.load_gather(vrows, [rows, col])
                        plsc.store_scatter(msgb, [rows, col], vv * exvs[h])
                return 0

            lax.fori_loop(0, B // L, g2_body, 0)
            pltpu.sync_copy(msgb, acc_sp.at[dsts_v.at[j]], add=True)
            return 0

        lax.fori_loop(0, BLKB, blk_body, 0)

        plsc.subcore_barrier()

        for off, sz in ((0, 128), (128, 128), (256, 128), (384, 128), (512, 113)):
            rs = rb + off
            pltpu.sync_copy(acc_sp.at[pl.ds(rs, sz)], vrows.at[pl.ds(0, sz)])
            pltpu.sync_copy(den_sp.at[pl.ds(rs, sz)], denb.at[pl.ds(0, sz)])
            def g3_body(g, _):
                rows = iota + g * L
                rvs = []
                for h in range(HQ):
                    dv = plsc.load_gather(
                        denb, [rows, jnp.full((L,), h, jnp.int32)])
                    rvs.append(1.0 / (dv + 1e-9))
                for d in range(DK):
                    for h in range(HQ):
                        col = jnp.broadcast_to(h * DK + d, (L,)).astype(jnp.int32)
                        av = plsc.load_gather(vrows, [rows, col])
                        plsc.store_scatter(obuf, [rows, col], av * rvs[h])
                return 0

            lax.fori_loop(0, B // L, g3_body, 0)
            pltpu.sync_copy(obuf.at[pl.ds(0, sz)], out_hbm.at[c, pl.ds(rs, sz)])

    return _agg_body


def _make_agg_call(hg):
    return functools.partial(
        pl.kernel,
        out_type=jax.ShapeDtypeStruct((NC, N, CQ), jnp.float32),
        mesh=_mesh,
        compiler_params=_sc_params,
        scratch_types=[
            pltpu.VMEM((BLKB, B), jnp.int32),
            pltpu.VMEM((BLKB, B), jnp.int32),
            pltpu.VMEM((HQ, EPT), jnp.float32),
            pltpu.VMEM((B, DEN_W), jnp.float32),
            pltpu.VMEM((B, CQ), jnp.float32),
            pltpu.VMEM((B, CQ), jnp.float32),
            pltpu.VMEM((B, CQ), jnp.float32),
            pltpu.VMEM((NW, L), jnp.float32),
            pltpu.VMEM((B, DEN_W), jnp.float32),
            pltpu.VMEM_SHARED((N, CQ), jnp.float32),
            pltpu.VMEM_SHARED((N, DEN_W), jnp.float32),
            pltpu.SemaphoreType.DMA,
        ],
    )(_make_agg_body(hg))


_agg_call_0 = _make_agg_call(0)
_agg_call_1 = _make_agg_call(1)


def _proj_ln_body(agg_ref, hd_ref, pw_ref, pb_ref, g_ref, b_ref, y_ref):
    dn = (((1,), (1,)), ((), ()))
    out = pb_ref[...][None, :]
    for qq in range(4):
        out = out + lax.dot_general(agg_ref[qq], pw_ref[:, qq * CQ:(qq + 1) * CQ],
                                    dn, preferred_element_type=jnp.float32)
    res = hd_ref[...] + out
    mu = jnp.mean(res, axis=-1, keepdims=True)
    var = jnp.mean((res - mu) * (res - mu), axis=-1, keepdims=True)
    y_ref[...] = (res - mu) * lax.rsqrt(var + 1e-5) * g_ref[...][None, :] + b_ref[...][None, :]


def _proj_ln(agg4, h_dst, proj_w, proj_b, ln_g, ln_b):
    grid = (N // ROW_BLK,)
    row_spec = pl.BlockSpec((ROW_BLK, DIM), lambda i: (i, 0))
    agg_spec = pl.BlockSpec((4, ROW_BLK, CQ), lambda i: (0, i, 0))
    w_spec = pl.BlockSpec((DIM, DIM), lambda i: (0, 0))
    b_spec = pl.BlockSpec((DIM,), lambda i: (0,))
    return pl.pallas_call(
        _proj_ln_body,
        grid=grid,
        in_specs=[agg_spec, row_spec, w_spec, b_spec, b_spec, b_spec],
        out_specs=row_spec,
        out_shape=jax.ShapeDtypeStruct((N, DIM), jnp.float32),
    )(agg4, h_dst, proj_w, proj_b, ln_g, ln_b)


def kernel(h_src, h_dst, edge_index, Wq_w, Wq_b, Wk_w, Wk_b, Wv_w, Wv_b,
           proj_w, proj_b, ln_g, ln_b):
    src = edge_index[0]
    dst = edge_index[1]
    padz = jnp.zeros((EPAD - E,), jnp.int32)
    src_p = jnp.concatenate([src, padz]).reshape(EPAD // B, B)
    dst_p = jnp.concatenate([dst, padz]).reshape(EPAD // B, B)

    q, k, v2 = _qkv(h_dst, h_src, Wq_w, Wk_w, Wv_w, Wq_b, Wk_b, Wv_b)

    score, wmax = _score_call(k, q, src_p, dst_p)

    zacc = jnp.zeros((N, CQ), jnp.float32)
    zden = jnp.zeros((N, DEN_W), jnp.float32)
    agg_a = _agg_call_0(score, wmax, v2, src_p, dst_p, zacc, zden)
    agg_b = _agg_call_1(score, wmax, v2, src_p, dst_p, zacc, zden)
    agg4 = jnp.concatenate([agg_a, agg_b], axis=0)

    return _proj_ln(agg4, h_dst, proj_w, proj_b, ln_g, ln_b)


# speedup vs baseline: 15.2580x; 2.8847x over previous
import functools
import numpy as np
import jax
import jax.numpy as jnp
from jax import lax
from jax.experimental import pallas as pl
from jax.experimental.pallas import tpu as pltpu
from jax.experimental.pallas import tpu_sc as plsc

N = 10000
E = 160000
DIM = 256
HEADS = 8
DK = DIM // HEADS

NC = 2
NS = 16
NW = NC * NS
L = 16

B = 128
EPAD = 163840
EPW = EPAD // NW
BLKA = EPW // B
EPT = EPAD // NS
BLKB = EPT // B
HH = HEADS // NC
CH = DIM // NC
HQ = 2
CQ = HQ * DK
DEN_W = 8
RPT = N // NS

ROW_BLK = 1000

_mesh = plsc.VectorSubcoreMesh(core_axis_name="c", subcore_axis_name="s")
_sc_params = pltpu.CompilerParams(use_tc_tiling_on_sc=False,
                                  needs_layout_passes=False)


def _qkv_body(hd_ref, hs_ref, wq_ref, wk_ref, wv_ref, bq_ref, bk_ref, bv_ref,
              q_ref, k_ref, v2_ref):
    dn = (((1,), (1,)), ((), ()))
    q = lax.dot_general(hd_ref[...], wq_ref[...], dn,
                        preferred_element_type=jnp.float32) + bq_ref[...][None, :]
    q_ref[...] = q * (1.0 / np.sqrt(DK))
    k_ref[...] = lax.dot_general(hs_ref[...], wk_ref[...], dn,
                                 preferred_element_type=jnp.float32) + bk_ref[...][None, :]
    v = lax.dot_general(hs_ref[...], wv_ref[...], dn,
                        preferred_element_type=jnp.float32) + bv_ref[...][None, :]
    for qq in range(4):
        v2_ref[qq] = v[:, qq * CQ:(qq + 1) * CQ]


def _qkv(h_dst, h_src, Wq_w, Wk_w, Wv_w, Wq_b, Wk_b, Wv_b):
    grid = (N // ROW_BLK,)
    row_spec = pl.BlockSpec((ROW_BLK, DIM), lambda i: (i, 0))
    w_spec = pl.BlockSpec((DIM, DIM), lambda i: (0, 0))
    b_spec = pl.BlockSpec((DIM,), lambda i: (0,))
    v2_spec = pl.BlockSpec((4, ROW_BLK, CQ), lambda i: (0, i, 0))
    return pl.pallas_call(
        _qkv_body,
        grid=grid,
        in_specs=[row_spec, row_spec, w_spec, w_spec, w_spec, b_spec, b_spec, b_spec],
        out_specs=[row_spec, row_spec, v2_spec],
        out_shape=[jax.ShapeDtypeStruct((N, DIM), jnp.float32),
                   jax.ShapeDtypeStruct((N, DIM), jnp.float32),
                   jax.ShapeDtypeStruct((4, N, CQ), jnp.float32)],
    )(h_dst, h_src, Wq_w, Wk_w, Wv_w, Wq_b, Wk_b, Wv_b)


def _score_body(k_hbm, q_hbm, src_hbm, dstq_hbm,
                score_hbm, wmax_hbm,
                src_v, dstq_v, krows, qrows, score_v, wbuf, semk, semq):
    w = lax.axis_index("s") * NC + lax.axis_index("c")
    pltpu.sync_copy(src_hbm.at[pl.ds(w * BLKA, BLKA)], src_v)
    pltpu.sync_copy(dstq_hbm.at[pl.ds(w * BLKA, BLKA)], dstq_v)

    iota = lax.iota(jnp.int32, L)

    def block_body(j, m):
        cpk = pltpu.async_copy(k_hbm.at[src_v.at[j]], krows, semk)
        cpq = pltpu.async_copy(q_hbm.at[dstq_v.at[j]], qrows, semq)
        cpk.wait()
        cpq.wait()

        def group_body(g, m):
            vecs = [jnp.zeros((L,), jnp.float32) for _ in range(HEADS)]
            for e in range(L):
                r = g * L + e
                prods = []
                for u in range(DIM // L):
                    pk = krows[r, pl.ds(u * L, L)]
                    pq = qrows[r, pl.ds(u * L, L)]
                    prods.append(pk * pq)
                lane = (iota == e)
                for h in range(HEADS):
                    s = jnp.sum(prods[2 * h] + prods[2 * h + 1])
                    vecs[h] = jnp.where(lane, s, vecs[h])
            off = j * B + g * L
            for h in range(HEADS):
                score_v[h, pl.ds(off, L)] = vecs[h]
                m = jnp.maximum(m, vecs[h])
            return m

        return lax.fori_loop(0, B // L, group_body, m)

    m = lax.fori_loop(0, BLKA, block_body, jnp.full((L,), -3e38, jnp.float32))
    wbuf[...] = m
    pltpu.sync_copy(wbuf, wmax_hbm.at[w])
    for h in range(HEADS):
        pltpu.sync_copy(score_v.at[h], score_hbm.at[h, pl.ds(w * EPW, EPW)])


_score_call = functools.partial(
    pl.kernel,
    out_type=[jax.ShapeDtypeStruct((HEADS, EPAD), jnp.float32),
              jax.ShapeDtypeStruct((NW, L), jnp.float32)],
    mesh=_mesh,
    compiler_params=_sc_params,
    scratch_types=[
        pltpu.VMEM((BLKA, B), jnp.int32),
        pltpu.VMEM((BLKA, B), jnp.int32),
        pltpu.VMEM((B, DIM), jnp.float32),
        pltpu.VMEM((B, DIM), jnp.float32),
        pltpu.VMEM((HEADS, EPW), jnp.float32),
        pltpu.VMEM((L,), jnp.float32),
        pltpu.SemaphoreType.DMA,
        pltpu.SemaphoreType.DMA,
    ],
)(_score_body)


def _make_agg_body(hg):
    def _agg_body(score_hbm, wmax_hbm, v2_hbm, src_hbm, dsts_hbm,
                  zacc_hbm, zden_hbm,
                  out_hbm,
                  src_v, dsts_v, ex_hm, exb, vrows, msgb, obuf, wmaxv, denb,
                  acc_sp, den_sp, semv):
        c = lax.axis_index("c")
        t = lax.axis_index("s")
        qq = hg * NC + c

        pltpu.sync_copy(wmax_hbm, wmaxv)
        m = wmaxv[0, :]
        for i in range(1, NW):
            m = jnp.maximum(m, wmaxv[i, :])
        gmax = jnp.max(m)

        rb = t * RPT
        pltpu.sync_copy(zacc_hbm.at[pl.ds(rb, RPT)], acc_sp.at[pl.ds(rb, RPT)])
        pltpu.sync_copy(zden_hbm.at[pl.ds(rb, RPT)], den_sp.at[pl.ds(rb, RPT)])

        pltpu.sync_copy(src_hbm.at[pl.ds(t * BLKB, BLKB)], src_v)
        pltpu.sync_copy(dsts_hbm.at[pl.ds(t * BLKB, BLKB)], dsts_v)

        iota = lax.iota(jnp.int32, L)
        ebase = t * EPT

        for g in range(B // L):
            rows = iota + g * L
            for h in range(DEN_W):
                plsc.store_scatter(exb, [rows, jnp.full((L,), h, jnp.int32)],
                                   jnp.zeros((L,), jnp.float32))

        for h in range(HQ):
            pltpu.sync_copy(score_hbm.at[qq * HQ + h, pl.ds(ebase, EPT)],
                            ex_hm.at[h])

            def ex_body(i, _, h=h):
                sv = ex_hm[h, pl.ds(i * L, L)]
                ev = jnp.exp(sv - gmax)
                gid = iota + (ebase + i * L)
                ev = jnp.where(gid < E, ev, 0.0)
                ex_hm[h, pl.ds(i * L, L)] = ev
                return 0

            lax.fori_loop(0, EPT // L, ex_body, 0, unroll=8)

        plsc.subcore_barrier()

        def den_body(j, _):
            def deng_body(g, _):
                rows = iota + g * L
                for h in range(HQ):
                    ev = ex_hm[h, pl.ds(j * B + g * L, L)]
                    plsc.store_scatter(
                        exb, [rows, jnp.full((L,), h, jnp.int32)], ev)
                return 0

            lax.fori_loop(0, B // L, deng_body, 0, unroll=4)
            pltpu.sync_copy(exb, den_sp.at[dsts_v.at[j]], add=True)
            return 0

        lax.fori_loop(0, BLKB, den_body, 0)

        def blk_body(j, _):
            pltpu.async_copy(v2_hbm.at[qq].at[src_v.at[j]], vrows, semv).wait()

            def g2_body(g, _):
                exvs = [ex_hm[h, pl.ds(j * B + g * L, L)] for h in range(HQ)]
                for e in range(L):
                    r = g * L + e
                    for h in range(HQ):
                        sc = jnp.broadcast_to(exvs[h][e], (L,))
                        for u in range(DK // L):
                            co = h * DK + u * L
                            msgb[r, pl.ds(co, L)] = vrows[r, pl.ds(co, L)] * sc
                return 0

            lax.fori_loop(0, B // L, g2_body, 0)
            pltpu.sync_copy(msgb, acc_sp.at[dsts_v.at[j]], add=True)
            return 0

        lax.fori_loop(0, BLKB, blk_body, 0)

        plsc.subcore_barrier()

        for off, sz in ((0, 128), (128, 128), (256, 128), (384, 128), (512, 113)):
            rs = rb + off
            pltpu.sync_copy(acc_sp.at[pl.ds(rs, sz)], vrows.at[pl.ds(0, sz)])
            pltpu.sync_copy(den_sp.at[pl.ds(rs, sz)], denb.at[pl.ds(0, sz)])
            def g3_body(g, _):
                rows = iota + g * L
                rvs = []
                for h in range(HQ):
                    dv = plsc.load_gather(
                        denb, [rows, jnp.full((L,), h, jnp.int32)])
                    rvs.append(1.0 / (dv + 1e-9))
                for e in range(L):
                    r = g * L + e
                    for h in range(HQ):
                        sc = jnp.broadcast_to(rvs[h][e], (L,))
                        for u in range(DK // L):
                            co = h * DK + u * L
                            obuf[r, pl.ds(co, L)] = vrows[r, pl.ds(co, L)] * sc
                return 0

            lax.fori_loop(0, B // L, g3_body, 0)
            pltpu.sync_copy(obuf.at[pl.ds(0, sz)], out_hbm.at[c, pl.ds(rs, sz)])

    return _agg_body


def _make_agg_call(hg):
    return functools.partial(
        pl.kernel,
        out_type=jax.ShapeDtypeStruct((NC, N, CQ), jnp.float32),
        mesh=_mesh,
        compiler_params=_sc_params,
        scratch_types=[
            pltpu.VMEM((BLKB, B), jnp.int32),
            pltpu.VMEM((BLKB, B), jnp.int32),
            pltpu.VMEM((HQ, EPT), jnp.float32),
            pltpu.VMEM((B, DEN_W), jnp.float32),
            pltpu.VMEM((B, CQ), jnp.float32),
            pltpu.VMEM((B, CQ), jnp.float32),
            pltpu.VMEM((B, CQ), jnp.float32),
            pltpu.VMEM((NW, L), jnp.float32),
            pltpu.VMEM((B, DEN_W), jnp.float32),
            pltpu.VMEM_SHARED((N, CQ), jnp.float32),
            pltpu.VMEM_SHARED((N, DEN_W), jnp.float32),
            pltpu.SemaphoreType.DMA,
        ],
    )(_make_agg_body(hg))


_agg_call_0 = _make_agg_call(0)
_agg_call_1 = _make_agg_call(1)


def _proj_ln_body(agg_ref, hd_ref, pw_ref, pb_ref, g_ref, b_ref, y_ref):
    dn = (((1,), (1,)), ((), ()))
    out = pb_ref[...][None, :]
    for qq in range(4):
        out = out + lax.dot_general(agg_ref[qq], pw_ref[:, qq * CQ:(qq + 1) * CQ],
                                    dn, preferred_element_type=jnp.float32)
    res = hd_ref[...] + out
    mu = jnp.mean(res, axis=-1, keepdims=True)
    var = jnp.mean((res - mu) * (res - mu), axis=-1, keepdims=True)
    y_ref[...] = (res - mu) * lax.rsqrt(var + 1e-5) * g_ref[...][None, :] + b_ref[...][None, :]


def _proj_ln(agg4, h_dst, proj_w, proj_b, ln_g, ln_b):
    grid = (N // ROW_BLK,)
    row_spec = pl.BlockSpec((ROW_BLK, DIM), lambda i: (i, 0))
    agg_spec = pl.BlockSpec((4, ROW_BLK, CQ), lambda i: (0, i, 0))
    w_spec = pl.BlockSpec((DIM, DIM), lambda i: (0, 0))
    b_spec = pl.BlockSpec((DIM,), lambda i: (0,))
    return pl.pallas_call(
        _proj_ln_body,
        grid=grid,
        in_specs=[agg_spec, row_spec, w_spec, b_spec, b_spec, b_spec],
        out_specs=row_spec,
        out_shape=jax.ShapeDtypeStruct((N, DIM), jnp.float32),
    )(agg4, h_dst, proj_w, proj_b, ln_g, ln_b)


def kernel(h_src, h_dst, edge_index, Wq_w, Wq_b, Wk_w, Wk_b, Wv_w, Wv_b,
           proj_w, proj_b, ln_g, ln_b):
    src = edge_index[0]
    dst = edge_index[1]
    padz = jnp.zeros((EPAD - E,), jnp.int32)
    src_p = jnp.concatenate([src, padz]).reshape(EPAD // B, B)
    dst_p = jnp.concatenate([dst, padz]).reshape(EPAD // B, B)

    q, k, v2 = _qkv(h_dst, h_src, Wq_w, Wk_w, Wv_w, Wq_b, Wk_b, Wv_b)

    score, wmax = _score_call(k, q, src_p, dst_p)

    zacc = jnp.zeros((N, CQ), jnp.float32)
    zden = jnp.zeros((N, DEN_W), jnp.float32)
    agg_a = _agg_call_0(score, wmax, v2, src_p, dst_p, zacc, zden)
    agg_b = _agg_call_1(score, wmax, v2, src_p, dst_p, zacc, zden)
    agg4 = jnp.concatenate([agg_a, agg_b], axis=0)

    return _proj_ln(agg4, h_dst, proj_w, proj_b, ln_g, ln_b)


# speedup vs baseline: 19.0487x; 1.2484x over previous
import functools
import numpy as np
import jax
import jax.numpy as jnp
from jax import lax
from jax.experimental import pallas as pl
from jax.experimental.pallas import tpu as pltpu
from jax.experimental.pallas import tpu_sc as plsc

N = 10000
E = 160000
DIM = 256
HEADS = 8
DK = DIM // HEADS

NC = 2
NS = 16
NW = NC * NS
L = 16

B = 128
BA = 80
NBLKA = 5120 // BA
FL = 16
EPAD = 163840
EPW = EPAD // NW
BLKA = EPW // B
EPT = EPAD // NS
BLKB = EPT // B
HH = HEADS // NC
CH = DIM // NC
HQ = 2
CQ = HQ * DK
DEN_W = 8
RPT = N // NS

ROW_BLK = 1000

_mesh = plsc.VectorSubcoreMesh(core_axis_name="c", subcore_axis_name="s")
_sc_params = pltpu.CompilerParams(use_tc_tiling_on_sc=False,
                                  needs_layout_passes=False)


def _qkv_body(hd_ref, hs_ref, wq_ref, wk_ref, wv_ref, bq_ref, bk_ref, bv_ref,
              q_ref, k_ref, v2_ref):
    dn = (((1,), (1,)), ((), ()))
    q = lax.dot_general(hd_ref[...], wq_ref[...], dn,
                        preferred_element_type=jnp.float32) + bq_ref[...][None, :]
    q_ref[...] = q * (1.0 / np.sqrt(DK))
    k_ref[...] = lax.dot_general(hs_ref[...], wk_ref[...], dn,
                                 preferred_element_type=jnp.float32) + bk_ref[...][None, :]
    v = lax.dot_general(hs_ref[...], wv_ref[...], dn,
                        preferred_element_type=jnp.float32) + bv_ref[...][None, :]
    for qq in range(4):
        v2_ref[qq] = v[:, qq * CQ:(qq + 1) * CQ]


def _qkv(h_dst, h_src, Wq_w, Wk_w, Wv_w, Wq_b, Wk_b, Wv_b):
    grid = (N // ROW_BLK,)
    row_spec = pl.BlockSpec((ROW_BLK, DIM), lambda i: (i, 0))
    w_spec = pl.BlockSpec((DIM, DIM), lambda i: (0, 0))
    b_spec = pl.BlockSpec((DIM,), lambda i: (0,))
    v2_spec = pl.BlockSpec((4, ROW_BLK, CQ), lambda i: (0, i, 0))
    return pl.pallas_call(
        _qkv_body,
        grid=grid,
        in_specs=[row_spec, row_spec, w_spec, w_spec, w_spec, b_spec, b_spec, b_spec],
        out_specs=[row_spec, row_spec, v2_spec],
        out_shape=[jax.ShapeDtypeStruct((N, DIM), jnp.float32),
                   jax.ShapeDtypeStruct((N, DIM), jnp.float32),
                   jax.ShapeDtypeStruct((4, N, CQ), jnp.float32)],
    )(h_dst, h_src, Wq_w, Wk_w, Wv_w, Wq_b, Wk_b, Wv_b)


def _score_body(k_hbm, q_hbm, src_hbm, dstq_hbm,
                score_hbm, wmax_hbm,
                src_v, dstq_v, krows0, qrows0, krows1, qrows1, scoreb, wbuf,
                semk0, semq0, semk1, semq1):
    w = lax.axis_index("s") * NC + lax.axis_index("c")
    pltpu.sync_copy(src_hbm.at[pl.ds(w * NBLKA, NBLKA)], src_v)
    pltpu.sync_copy(dstq_hbm.at[pl.ds(w * NBLKA, NBLKA)], dstq_v)

    iota = lax.iota(jnp.int32, L)
    kbufs = (krows0, krows1)
    qbufs = (qrows0, qrows1)
    ksems = (semk0, semk1)
    qsems = (semq0, semq1)

    pltpu.async_copy(k_hbm.at[src_v.at[0]], krows0, semk0)
    pltpu.async_copy(q_hbm.at[dstq_v.at[0]], qrows0, semq0)

    def block_step(j, m, kb, qb, ks, qs, nkb, nqb, nks, nqs):
        @pl.when(j + 1 < NBLKA)
        def _():
            pltpu.async_copy(k_hbm.at[src_v.at[j + 1]], nkb, nks)
            pltpu.async_copy(q_hbm.at[dstq_v.at[j + 1]], nqb, nqs)

        pltpu.make_async_copy(k_hbm.at[src_v.at[j]], kb, ks).wait()
        pltpu.make_async_copy(q_hbm.at[dstq_v.at[j]], qb, qs).wait()

        coff = (j % FL) * BA

        def group_body(g, m):
            vecs = [jnp.zeros((L,), jnp.float32) for _ in range(HEADS)]
            for e in range(L):
                r = g * L + e
                prods = []
                for u in range(DIM // L):
                    pk = kb[r, pl.ds(u * L, L)]
                    pq = qb[r, pl.ds(u * L, L)]
                    prods.append(pk * pq)
                lane = (iota == e)
                for h in range(HEADS):
                    s = jnp.sum(prods[2 * h] + prods[2 * h + 1])
                    vecs[h] = jnp.where(lane, s, vecs[h])
            for h in range(HEADS):
                scoreb[h, pl.ds(coff + g * L, L)] = vecs[h]
                m = jnp.maximum(m, vecs[h])
            return m

        m = lax.fori_loop(0, BA // L, group_body, m)

        @pl.when(j % FL == FL - 1)
        def _():
            fbase = w * EPW + (j // FL) * (FL * BA)
            for h in range(HEADS):
                pltpu.sync_copy(scoreb.at[h],
                                score_hbm.at[h, pl.ds(fbase, FL * BA)])

        return m

    def pair_body(jo, m):
        m = block_step(2 * jo, m, krows0, qrows0, semk0, semq0,
                       krows1, qrows1, semk1, semq1)
        m = block_step(2 * jo + 1, m, krows1, qrows1, semk1, semq1,
                       krows0, qrows0, semk0, semq0)
        return m

    m = lax.fori_loop(0, NBLKA // 2, pair_body,
                      jnp.full((L,), -3e38, jnp.float32))
    wbuf[...] = m
    pltpu.sync_copy(wbuf, wmax_hbm.at[w])


_score_call = functools.partial(
    pl.kernel,
    out_type=[jax.ShapeDtypeStruct((HEADS, EPAD), jnp.float32),
              jax.ShapeDtypeStruct((NW, L), jnp.float32)],
    mesh=_mesh,
    compiler_params=_sc_params,
    scratch_types=[
        pltpu.VMEM((NBLKA, BA), jnp.int32),
        pltpu.VMEM((NBLKA, BA), jnp.int32),
        pltpu.VMEM((BA, DIM), jnp.float32),
        pltpu.VMEM((BA, DIM), jnp.float32),
        pltpu.VMEM((BA, DIM), jnp.float32),
        pltpu.VMEM((BA, DIM), jnp.float32),
        pltpu.VMEM((HEADS, FL * BA), jnp.float32),
        pltpu.VMEM((L,), jnp.float32),
        pltpu.SemaphoreType.DMA,
        pltpu.SemaphoreType.DMA,
        pltpu.SemaphoreType.DMA,
        pltpu.SemaphoreType.DMA,
    ],
)(_score_body)


def _make_agg_body(hg):
    def _agg_body(score_hbm, wmax_hbm, v2_hbm, src_hbm, dsts_hbm,
                  zacc_hbm, zden_hbm,
                  out_hbm,
                  src_v, dsts_v, ex_hm, exb, vrows, msgb, obuf, wmaxv, denb,
                  acc_sp, den_sp, semv):
        c = lax.axis_index("c")
        t = lax.axis_index("s")
        qq = hg * NC + c

        pltpu.sync_copy(wmax_hbm, wmaxv)
        m = wmaxv[0, :]
        for i in range(1, NW):
            m = jnp.maximum(m, wmaxv[i, :])
        gmax = jnp.max(m)

        rb = t * RPT
        pltpu.sync_copy(zacc_hbm.at[pl.ds(rb, RPT)], acc_sp.at[pl.ds(rb, RPT)])
        pltpu.sync_copy(zden_hbm.at[pl.ds(rb, RPT)], den_sp.at[pl.ds(rb, RPT)])

        pltpu.sync_copy(src_hbm.at[pl.ds(t * BLKB, BLKB)], src_v)
        pltpu.sync_copy(dsts_hbm.at[pl.ds(t * BLKB, BLKB)], dsts_v)

        iota = lax.iota(jnp.int32, L)
        ebase = t * EPT

        for g in range(B // L):
            rows = iota + g * L
            for h in range(DEN_W):
                plsc.store_scatter(exb, [rows, jnp.full((L,), h, jnp.int32)],
                                   jnp.zeros((L,), jnp.float32))

        for h in range(HQ):
            pltpu.sync_copy(score_hbm.at[qq * HQ + h, pl.ds(ebase, EPT)],
                            ex_hm.at[h])

            def ex_body(i, _, h=h):
                sv = ex_hm[h, pl.ds(i * L, L)]
                ev = jnp.exp(sv - gmax)
                gid = iota + (ebase + i * L)
                ev = jnp.where(gid < E, ev, 0.0)
                ex_hm[h, pl.ds(i * L, L)] = ev
                return 0

            lax.fori_loop(0, EPT // L, ex_body, 0, unroll=8)

        plsc.subcore_barrier()

        def den_body(j, _):
            def deng_body(g, _):
                rows = iota + g * L
                for h in range(HQ):
                    ev = ex_hm[h, pl.ds(j * B + g * L, L)]
                    plsc.store_scatter(
                        exb, [rows, jnp.full((L,), h, jnp.int32)], ev)
                return 0

            lax.fori_loop(0, B // L, deng_body, 0, unroll=4)
            pltpu.sync_copy(exb, den_sp.at[dsts_v.at[j]], add=True)
            return 0

        lax.fori_loop(0, BLKB, den_body, 0)

        def blk_body(j, _):
            pltpu.async_copy(v2_hbm.at[qq].at[src_v.at[j]], vrows, semv).wait()

            def g2_body(g, _):
                exvs = [ex_hm[h, pl.ds(j * B + g * L, L)] for h in range(HQ)]
                for e in range(L):
                    r = g * L + e
                    for h in range(HQ):
                        sc = jnp.broadcast_to(exvs[h][e], (L,))
                        for u in range(DK // L):
                            co = h * DK + u * L
                            msgb[r, pl.ds(co, L)] = vrows[r, pl.ds(co, L)] * sc
                return 0

            lax.fori_loop(0, B // L, g2_body, 0)
            pltpu.sync_copy(msgb, acc_sp.at[dsts_v.at[j]], add=True)
            return 0

        lax.fori_loop(0, BLKB, blk_body, 0)

        plsc.subcore_barrier()

        for off, sz in ((0, 128), (128, 128), (256, 128), (384, 128), (512, 113)):
            rs = rb + off
            pltpu.sync_copy(acc_sp.at[pl.ds(rs, sz)], vrows.at[pl.ds(0, sz)])
            pltpu.sync_copy(den_sp.at[pl.ds(rs, sz)], denb.at[pl.ds(0, sz)])
            def g3_body(g, _):
                rows = iota + g * L
                rvs = []
                for h in range(HQ):
                    dv = plsc.load_gather(
                        denb, [rows, jnp.full((L,), h, jnp.int32)])
                    rvs.append(1.0 / (dv + 1e-9))
                for e in range(L):
                    r = g * L + e
                    for h in range(HQ):
                        sc = jnp.broadcast_to(rvs[h][e], (L,))
                        for u in range(DK // L):
                            co = h * DK + u * L
                            obuf[r, pl.ds(co, L)] = vrows[r, pl.ds(co, L)] * sc
                return 0

            lax.fori_loop(0, B // L, g3_body, 0)
            pltpu.sync_copy(obuf.at[pl.ds(0, sz)], out_hbm.at[c, pl.ds(rs, sz)])

    return _agg_body


def _make_agg_call(hg):
    return functools.partial(
        pl.kernel,
        out_type=jax.ShapeDtypeStruct((NC, N, CQ), jnp.float32),
        mesh=_mesh,
        compiler_params=_sc_params,
        scratch_types=[
            pltpu.VMEM((BLKB, B), jnp.int32),
            pltpu.VMEM((BLKB, B), jnp.int32),
            pltpu.VMEM((HQ, EPT), jnp.float32),
            pltpu.VMEM((B, DEN_W), jnp.float32),
            pltpu.VMEM((B, CQ), jnp.float32),
            pltpu.VMEM((B, CQ), jnp.float32),
            pltpu.VMEM((B, CQ), jnp.float32),
            pltpu.VMEM((NW, L), jnp.float32),
            pltpu.VMEM((B, DEN_W), jnp.float32),
            pltpu.VMEM_SHARED((N, CQ), jnp.float32),
            pltpu.VMEM_SHARED((N, DEN_W), jnp.float32),
            pltpu.SemaphoreType.DMA,
        ],
    )(_make_agg_body(hg))


_agg_call_0 = _make_agg_call(0)
_agg_call_1 = _make_agg_call(1)


def _proj_ln_body(agg_ref, hd_ref, pw_ref, pb_ref, g_ref, b_ref, y_ref):
    dn = (((1,), (1,)), ((), ()))
    out = pb_ref[...][None, :]
    for qq in range(4):
        out = out + lax.dot_general(agg_ref[qq], pw_ref[:, qq * CQ:(qq + 1) * CQ],
                                    dn, preferred_element_type=jnp.float32)
    res = hd_ref[...] + out
    mu = jnp.mean(res, axis=-1, keepdims=True)
    var = jnp.mean((res - mu) * (res - mu), axis=-1, keepdims=True)
    y_ref[...] = (res - mu) * lax.rsqrt(var + 1e-5) * g_ref[...][None, :] + b_ref[...][None, :]


def _proj_ln(agg4, h_dst, proj_w, proj_b, ln_g, ln_b):
    grid = (N // ROW_BLK,)
    row_spec = pl.BlockSpec((ROW_BLK, DIM), lambda i: (i, 0))
    agg_spec = pl.BlockSpec((4, ROW_BLK, CQ), lambda i: (0, i, 0))
    w_spec = pl.BlockSpec((DIM, DIM), lambda i: (0, 0))
    b_spec = pl.BlockSpec((DIM,), lambda i: (0,))
    return pl.pallas_call(
        _proj_ln_body,
        grid=grid,
        in_specs=[agg_spec, row_spec, w_spec, b_spec, b_spec, b_spec],
        out_specs=row_spec,
        out_shape=jax.ShapeDtypeStruct((N, DIM), jnp.float32),
    )(agg4, h_dst, proj_w, proj_b, ln_g, ln_b)


def kernel(h_src, h_dst, edge_index, Wq_w, Wq_b, Wk_w, Wk_b, Wv_w, Wv_b,
           proj_w, proj_b, ln_g, ln_b):
    src = edge_index[0]
    dst = edge_index[1]
    padz = jnp.zeros((EPAD - E,), jnp.int32)
    src_f = jnp.concatenate([src, padz])
    dst_f = jnp.concatenate([dst, padz])
    src_p = src_f.reshape(EPAD // B, B)
    dst_p = dst_f.reshape(EPAD // B, B)
    src_a = src_f.reshape(EPAD // BA, BA)
    dst_a = dst_f.reshape(EPAD // BA, BA)

    q, k, v2 = _qkv(h_dst, h_src, Wq_w, Wk_w, Wv_w, Wq_b, Wk_b, Wv_b)

    score, wmax = _score_call(k, q, src_a, dst_a)

    zacc = jnp.zeros((N, CQ), jnp.float32)
    zden = jnp.zeros((N, DEN_W), jnp.float32)
    agg_a = _agg_call_0(score, wmax, v2, src_p, dst_p, zacc, zden)
    agg_b = _agg_call_1(score, wmax, v2, src_p, dst_p, zacc, zden)
    agg4 = jnp.concatenate([agg_a, agg_b], axis=0)

    return _proj_ln(agg4, h_dst, proj_w, proj_b, ln_g, ln_b)


# speedup vs baseline: 21.5058x; 1.1290x over previous
import functools
import numpy as np
import jax
import jax.numpy as jnp
from jax import lax
from jax.experimental import pallas as pl
from jax.experimental.pallas import tpu as pltpu
from jax.experimental.pallas import tpu_sc as plsc

N = 10000
E = 160000
DIM = 256
HEADS = 8
DK = DIM // HEADS

NC = 2
NS = 16
NW = NC * NS
L = 16

B = 128
BA = 80
NBLKA = 5120 // BA
FL = 16
EPAD = 163840
EPW = EPAD // NW
BLKA = EPW // B
EPT = EPAD // NS
BLKB = EPT // B
HH = HEADS // NC
CH = DIM // NC
HQ = 2
CQ = HQ * DK
DEN_W = 8
RPT = N // NS

ROW_BLK = 1000

_mesh = plsc.VectorSubcoreMesh(core_axis_name="c", subcore_axis_name="s")
_sc_params = pltpu.CompilerParams(use_tc_tiling_on_sc=False,
                                  needs_layout_passes=False)


def _qkv_body(hd_ref, hs_ref, wq_ref, wk_ref, wv_ref, bq_ref, bk_ref, bv_ref,
              q_ref, k_ref, v2_ref):
    dn = (((1,), (1,)), ((), ()))
    q = lax.dot_general(hd_ref[...], wq_ref[...], dn,
                        preferred_element_type=jnp.float32) + bq_ref[...][None, :]
    q_ref[...] = q * (1.0 / np.sqrt(DK))
    k_ref[...] = lax.dot_general(hs_ref[...], wk_ref[...], dn,
                                 preferred_element_type=jnp.float32) + bk_ref[...][None, :]
    v = lax.dot_general(hs_ref[...], wv_ref[...], dn,
                        preferred_element_type=jnp.float32) + bv_ref[...][None, :]
    for qq in range(4):
        v2_ref[qq] = v[:, qq * CQ:(qq + 1) * CQ]


def _qkv(h_dst, h_src, Wq_w, Wk_w, Wv_w, Wq_b, Wk_b, Wv_b):
    grid = (N // ROW_BLK,)
    row_spec = pl.BlockSpec((ROW_BLK, DIM), lambda i: (i, 0))
    w_spec = pl.BlockSpec((DIM, DIM), lambda i: (0, 0))
    b_spec = pl.BlockSpec((DIM,), lambda i: (0,))
    v2_spec = pl.BlockSpec((4, ROW_BLK, CQ), lambda i: (0, i, 0))
    return pl.pallas_call(
        _qkv_body,
        grid=grid,
        in_specs=[row_spec, row_spec, w_spec, w_spec, w_spec, b_spec, b_spec, b_spec],
        out_specs=[row_spec, row_spec, v2_spec],
        out_shape=[jax.ShapeDtypeStruct((N, DIM), jnp.float32),
                   jax.ShapeDtypeStruct((N, DIM), jnp.float32),
                   jax.ShapeDtypeStruct((4, N, CQ), jnp.float32)],
    )(h_dst, h_src, Wq_w, Wk_w, Wv_w, Wq_b, Wk_b, Wv_b)


def _score_body(k_hbm, q_hbm, src_hbm, dstq_hbm,
                score_hbm, wmax_hbm,
                src_v, dstq_v, krows0, qrows0, krows1, qrows1, scoreb, wbuf,
                semk0, semq0, semk1, semq1):
    w = lax.axis_index("s") * NC + lax.axis_index("c")
    pltpu.sync_copy(src_hbm.at[pl.ds(w * NBLKA, NBLKA)], src_v)
    pltpu.sync_copy(dstq_hbm.at[pl.ds(w * NBLKA, NBLKA)], dstq_v)

    iota = lax.iota(jnp.int32, L)
    kbufs = (krows0, krows1)
    qbufs = (qrows0, qrows1)
    ksems = (semk0, semk1)
    qsems = (semq0, semq1)

    pltpu.async_copy(k_hbm.at[src_v.at[0]], krows0, semk0)
    pltpu.async_copy(q_hbm.at[dstq_v.at[0]], qrows0, semq0)

    def block_step(j, m, kb, qb, ks, qs, nkb, nqb, nks, nqs):
        @pl.when(j + 1 < NBLKA)
        def _():
            pltpu.async_copy(k_hbm.at[src_v.at[j + 1]], nkb, nks)
            pltpu.async_copy(q_hbm.at[dstq_v.at[j + 1]], nqb, nqs)

        pltpu.make_async_copy(k_hbm.at[src_v.at[j]], kb, ks).wait()
        pltpu.make_async_copy(q_hbm.at[dstq_v.at[j]], qb, qs).wait()

        coff = (j % FL) * BA

        def group_body(g, m):
            vecs = [jnp.zeros((L,), jnp.float32) for _ in range(HEADS)]
            for e in range(L):
                r = g * L + e
                prods = []
                for u in range(DIM // L):
                    pk = kb[r, pl.ds(u * L, L)]
                    pq = qb[r, pl.ds(u * L, L)]
                    prods.append(pk * pq)
                lane = (iota == e)
                for h in range(HEADS):
                    s = jnp.sum(prods[2 * h] + prods[2 * h + 1])
                    vecs[h] = jnp.where(lane, s, vecs[h])
            for h in range(HEADS):
                scoreb[h, pl.ds(coff + g * L, L)] = vecs[h]
                m = jnp.maximum(m, vecs[h])
            return m

        m = lax.fori_loop(0, BA // L, group_body, m)

        @pl.when(j % FL == FL - 1)
        def _():
            fbase = w * EPW + (j // FL) * (FL * BA)
            for h in range(HEADS):
                pltpu.sync_copy(scoreb.at[h],
                                score_hbm.at[h, pl.ds(fbase, FL * BA)])

        return m

    def pair_body(jo, m):
        m = block_step(2 * jo, m, krows0, qrows0, semk0, semq0,
                       krows1, qrows1, semk1, semq1)
        m = block_step(2 * jo + 1, m, krows1, qrows1, semk1, semq1,
                       krows0, qrows0, semk0, semq0)
        return m

    m = lax.fori_loop(0, NBLKA // 2, pair_body,
                      jnp.full((L,), -3e38, jnp.float32))
    wbuf[...] = m
    pltpu.sync_copy(wbuf, wmax_hbm.at[w])


_score_call = functools.partial(
    pl.kernel,
    out_type=[jax.ShapeDtypeStruct((HEADS, EPAD), jnp.float32),
              jax.ShapeDtypeStruct((NW, L), jnp.float32)],
    mesh=_mesh,
    compiler_params=_sc_params,
    scratch_types=[
        pltpu.VMEM((NBLKA, BA), jnp.int32),
        pltpu.VMEM((NBLKA, BA), jnp.int32),
        pltpu.VMEM((BA, DIM), jnp.float32),
        pltpu.VMEM((BA, DIM), jnp.float32),
        pltpu.VMEM((BA, DIM), jnp.float32),
        pltpu.VMEM((BA, DIM), jnp.float32),
        pltpu.VMEM((HEADS, FL * BA), jnp.float32),
        pltpu.VMEM((L,), jnp.float32),
        pltpu.SemaphoreType.DMA,
        pltpu.SemaphoreType.DMA,
        pltpu.SemaphoreType.DMA,
        pltpu.SemaphoreType.DMA,
    ],
)(_score_body)


def _make_agg_body(hg):
    def _agg_body(score_hbm, wmax_hbm, v2_hbm, src_hbm, dsts_hbm,
                  zacc_hbm, zden_hbm,
                  out_hbm,
                  src_v, dsts_v, ex_hm, exb, vrows0, vrows1, msgb, obuf,
                  wmaxv, denb, acc_sp, den_sp, semv0, semv1):
        c = lax.axis_index("c")
        t = lax.axis_index("s")
        qq = hg * NC + c

        pltpu.sync_copy(wmax_hbm, wmaxv)
        m = wmaxv[0, :]
        for i in range(1, NW):
            m = jnp.maximum(m, wmaxv[i, :])
        gmax = jnp.max(m)

        rb = t * RPT
        pltpu.sync_copy(zacc_hbm.at[pl.ds(rb, RPT)], acc_sp.at[pl.ds(rb, RPT)])
        pltpu.sync_copy(zden_hbm.at[pl.ds(rb, RPT)], den_sp.at[pl.ds(rb, RPT)])

        pltpu.sync_copy(src_hbm.at[pl.ds(t * BLKB, BLKB)], src_v)
        pltpu.sync_copy(dsts_hbm.at[pl.ds(t * BLKB, BLKB)], dsts_v)

        iota = lax.iota(jnp.int32, L)
        ebase = t * EPT

        for g in range(B // L):
            rows = iota + g * L
            for h in range(DEN_W):
                plsc.store_scatter(exb, [rows, jnp.full((L,), h, jnp.int32)],
                                   jnp.zeros((L,), jnp.float32))

        for h in range(HQ):
            pltpu.sync_copy(score_hbm.at[qq * HQ + h, pl.ds(ebase, EPT)],
                            ex_hm.at[h])

            def ex_body(i, _, h=h):
                sv = ex_hm[h, pl.ds(i * L, L)]
                ev = jnp.exp(sv - gmax)
                gid = iota + (ebase + i * L)
                ev = jnp.where(gid < E, ev, 0.0)
                ex_hm[h, pl.ds(i * L, L)] = ev
                return 0

            lax.fori_loop(0, EPT // L, ex_body, 0, unroll=8)

        plsc.subcore_barrier()

        def den_body(j, _):
            def deng_body(g, _):
                rows = iota + g * L
                for h in range(HQ):
                    ev = ex_hm[h, pl.ds(j * B + g * L, L)]
                    plsc.store_scatter(
                        exb, [rows, jnp.full((L,), h, jnp.int32)], ev)
                return 0

            lax.fori_loop(0, B // L, deng_body, 0, unroll=4)
            pltpu.sync_copy(exb, den_sp.at[dsts_v.at[j]], add=True)
            return 0

        lax.fori_loop(0, BLKB, den_body, 0)

        pltpu.async_copy(v2_hbm.at[qq].at[src_v.at[0]], vrows0, semv0)

        def blk_step(j, vb, vs, nvb, nvs):
            @pl.when(j + 1 < BLKB)
            def _():
                pltpu.async_copy(v2_hbm.at[qq].at[src_v.at[j + 1]], nvb, nvs)

            pltpu.make_async_copy(v2_hbm.at[qq].at[src_v.at[j]], vb, vs).wait()

            def g2_body(g, _):
                exvs = [ex_hm[h, pl.ds(j * B + g * L, L)] for h in range(HQ)]
                for e in range(L):
                    r = g * L + e
                    for h in range(HQ):
                        sc = jnp.broadcast_to(exvs[h][e], (L,))
                        for u in range(DK // L):
                            co = h * DK + u * L
                            msgb[r, pl.ds(co, L)] = vb[r, pl.ds(co, L)] * sc
                return 0

            lax.fori_loop(0, B // L, g2_body, 0)
            pltpu.sync_copy(msgb, acc_sp.at[dsts_v.at[j]], add=True)

        def pair2_body(jo, _):
            blk_step(2 * jo, vrows0, semv0, vrows1, semv1)
            blk_step(2 * jo + 1, vrows1, semv1, vrows0, semv0)
            return 0

        lax.fori_loop(0, BLKB // 2, pair2_body, 0)

        plsc.subcore_barrier()

        for off, sz in ((0, 128), (128, 128), (256, 128), (384, 128), (512, 113)):
            rs = rb + off
            pltpu.sync_copy(acc_sp.at[pl.ds(rs, sz)], vrows0.at[pl.ds(0, sz)])
            pltpu.sync_copy(den_sp.at[pl.ds(rs, sz)], denb.at[pl.ds(0, sz)])
            def g3_body(g, _):
                rows = iota + g * L
                rvs = []
                for h in range(HQ):
                    dv = plsc.load_gather(
                        denb, [rows, jnp.full((L,), h, jnp.int32)])
                    rvs.append(1.0 / (dv + 1e-9))
                for e in range(L):
                    r = g * L + e
                    for h in range(HQ):
                        sc = jnp.broadcast_to(rvs[h][e], (L,))
                        for u in range(DK // L):
                            co = h * DK + u * L
                            obuf[r, pl.ds(co, L)] = vrows0[r, pl.ds(co, L)] * sc
                return 0

            lax.fori_loop(0, B // L, g3_body, 0)
            pltpu.sync_copy(obuf.at[pl.ds(0, sz)], out_hbm.at[c, pl.ds(rs, sz)])

    return _agg_body


def _make_agg_call(hg):
    return functools.partial(
        pl.kernel,
        out_type=jax.ShapeDtypeStruct((NC, N, CQ), jnp.float32),
        mesh=_mesh,
        compiler_params=_sc_params,
        scratch_types=[
            pltpu.VMEM((BLKB, B), jnp.int32),
            pltpu.VMEM((BLKB, B), jnp.int32),
            pltpu.VMEM((HQ, EPT), jnp.float32),
            pltpu.VMEM((B, DEN_W), jnp.float32),
            pltpu.VMEM((B, CQ), jnp.float32),
            pltpu.VMEM((B, CQ), jnp.float32),
            pltpu.VMEM((B, CQ), jnp.float32),
            pltpu.VMEM((B, CQ), jnp.float32),
            pltpu.VMEM((NW, L), jnp.float32),
            pltpu.VMEM((B, DEN_W), jnp.float32),
            pltpu.VMEM_SHARED((N, CQ), jnp.float32),
            pltpu.VMEM_SHARED((N, DEN_W), jnp.float32),
            pltpu.SemaphoreType.DMA,
            pltpu.SemaphoreType.DMA,
        ],
    )(_make_agg_body(hg))


_agg_call_0 = _make_agg_call(0)
_agg_call_1 = _make_agg_call(1)


def _proj_ln_body(agg_ref, hd_ref, pw_ref, pb_ref, g_ref, b_ref, y_ref):
    dn = (((1,), (1,)), ((), ()))
    out = pb_ref[...][None, :]
    for qq in range(4):
        out = out + lax.dot_general(agg_ref[qq], pw_ref[:, qq * CQ:(qq + 1) * CQ],
                                    dn, preferred_element_type=jnp.float32)
    res = hd_ref[...] + out
    mu = jnp.mean(res, axis=-1, keepdims=True)
    var = jnp.mean((res - mu) * (res - mu), axis=-1, keepdims=True)
    y_ref[...] = (res - mu) * lax.rsqrt(var + 1e-5) * g_ref[...][None, :] + b_ref[...][None, :]


def _proj_ln(agg4, h_dst, proj_w, proj_b, ln_g, ln_b):
    grid = (N // ROW_BLK,)
    row_spec = pl.BlockSpec((ROW_BLK, DIM), lambda i: (i, 0))
    agg_spec = pl.BlockSpec((4, ROW_BLK, CQ), lambda i: (0, i, 0))
    w_spec = pl.BlockSpec((DIM, DIM), lambda i: (0, 0))
    b_spec = pl.BlockSpec((DIM,), lambda i: (0,))
    return pl.pallas_call(
        _proj_ln_body,
        grid=grid,
        in_specs=[agg_spec, row_spec, w_spec, b_spec, b_spec, b_spec],
        out_specs=row_spec,
        out_shape=jax.ShapeDtypeStruct((N, DIM), jnp.float32),
    )(agg4, h_dst, proj_w, proj_b, ln_g, ln_b)


def kernel(h_src, h_dst, edge_index, Wq_w, Wq_b, Wk_w, Wk_b, Wv_w, Wv_b,
           proj_w, proj_b, ln_g, ln_b):
    src = edge_index[0]
    dst = edge_index[1]
    padz = jnp.zeros((EPAD - E,), jnp.int32)
    src_f = jnp.concatenate([src, padz])
    dst_f = jnp.concatenate([dst, padz])
    src_p = src_f.reshape(EPAD // B, B)
    dst_p = dst_f.reshape(EPAD // B, B)
    src_a = src_f.reshape(EPAD // BA, BA)
    dst_a = dst_f.reshape(EPAD // BA, BA)

    q, k, v2 = _qkv(h_dst, h_src, Wq_w, Wk_w, Wv_w, Wq_b, Wk_b, Wv_b)

    score, wmax = _score_call(k, q, src_a, dst_a)

    zacc = jnp.zeros((N, CQ), jnp.float32)
    zden = jnp.zeros((N, DEN_W), jnp.float32)
    agg_a = _agg_call_0(score, wmax, v2, src_p, dst_p, zacc, zden)
    agg_b = _agg_call_1(score, wmax, v2, src_p, dst_p, zacc, zden)
    agg4 = jnp.concatenate([agg_a, agg_b], axis=0)

    return _proj_ln(agg4, h_dst, proj_w, proj_b, ln_g, ln_b)


# speedup vs baseline: 21.7598x; 1.0118x over previous
import functools
import numpy as np
import jax
import jax.numpy as jnp
from jax import lax
from jax.experimental import pallas as pl
from jax.experimental.pallas import tpu as pltpu
from jax.experimental.pallas import tpu_sc as plsc

N = 10000
E = 160000
DIM = 256
HEADS = 8
DK = DIM // HEADS

NC = 2
NS = 16
NW = NC * NS
L = 16

B = 128
BA = 80
NBLKA = 5120 // BA
FL = 16
EPAD = 163840
EPW = EPAD // NW
BLKA = EPW // B
EPT = EPAD // NS
BLKB = EPT // B
HH = HEADS // NC
CH = DIM // NC
HQ = 2
CQ = HQ * DK
DEN_W = 8
RPT = N // NS

ROW_BLK = 1000

_mesh = plsc.VectorSubcoreMesh(core_axis_name="c", subcore_axis_name="s")
_sc_params = pltpu.CompilerParams(use_tc_tiling_on_sc=False,
                                  needs_layout_passes=False)


def _qkv_body(hd_ref, hs_ref, wq_ref, wk_ref, wv_ref, bq_ref, bk_ref, bv_ref,
              q_ref, k_ref, v2_ref):
    dn = (((1,), (1,)), ((), ()))
    q = lax.dot_general(hd_ref[...], wq_ref[...], dn,
                        preferred_element_type=jnp.float32) + bq_ref[...][None, :]
    q_ref[...] = q * (1.0 / np.sqrt(DK))
    k_ref[...] = lax.dot_general(hs_ref[...], wk_ref[...], dn,
                                 preferred_element_type=jnp.float32) + bk_ref[...][None, :]
    v = lax.dot_general(hs_ref[...], wv_ref[...], dn,
                        preferred_element_type=jnp.float32) + bv_ref[...][None, :]
    for qq in range(4):
        v2_ref[qq] = v[:, qq * CQ:(qq + 1) * CQ]


def _qkv(h_dst, h_src, Wq_w, Wk_w, Wv_w, Wq_b, Wk_b, Wv_b):
    grid = (N // ROW_BLK,)
    row_spec = pl.BlockSpec((ROW_BLK, DIM), lambda i: (i, 0))
    w_spec = pl.BlockSpec((DIM, DIM), lambda i: (0, 0))
    b_spec = pl.BlockSpec((DIM,), lambda i: (0,))
    v2_spec = pl.BlockSpec((4, ROW_BLK, CQ), lambda i: (0, i, 0))
    return pl.pallas_call(
        _qkv_body,
        grid=grid,
        in_specs=[row_spec, row_spec, w_spec, w_spec, w_spec, b_spec, b_spec, b_spec],
        out_specs=[row_spec, row_spec, v2_spec],
        out_shape=[jax.ShapeDtypeStruct((N, DIM), jnp.float32),
                   jax.ShapeDtypeStruct((N, DIM), jnp.float32),
                   jax.ShapeDtypeStruct((4, N, CQ), jnp.float32)],
    )(h_dst, h_src, Wq_w, Wk_w, Wv_w, Wq_b, Wk_b, Wv_b)


def _score_body(k_hbm, q_hbm, src_hbm, dstq_hbm,
                score_hbm, wmax_hbm,
                src_v, dstq_v, krows0, qrows0, krows1, qrows1, scoreb, wbuf,
                semk0, semq0, semk1, semq1):
    w = lax.axis_index("s") * NC + lax.axis_index("c")
    pltpu.sync_copy(src_hbm.at[pl.ds(w * NBLKA, NBLKA)], src_v)
    pltpu.sync_copy(dstq_hbm.at[pl.ds(w * NBLKA, NBLKA)], dstq_v)

    iota = lax.iota(jnp.int32, L)
    kbufs = (krows0, krows1)
    qbufs = (qrows0, qrows1)
    ksems = (semk0, semk1)
    qsems = (semq0, semq1)

    pltpu.async_copy(k_hbm.at[src_v.at[0]], krows0, semk0)
    pltpu.async_copy(q_hbm.at[dstq_v.at[0]], qrows0, semq0)

    def block_step(j, m, kb, qb, ks, qs, nkb, nqb, nks, nqs):
        @pl.when(j + 1 < NBLKA)
        def _():
            pltpu.async_copy(k_hbm.at[src_v.at[j + 1]], nkb, nks)
            pltpu.async_copy(q_hbm.at[dstq_v.at[j + 1]], nqb, nqs)

        pltpu.make_async_copy(k_hbm.at[src_v.at[j]], kb, ks).wait()
        pltpu.make_async_copy(q_hbm.at[dstq_v.at[j]], qb, qs).wait()

        coff = (j % FL) * BA

        def group_body(g, m):
            vecs = [jnp.zeros((L,), jnp.float32) for _ in range(HEADS)]
            for e in range(L):
                r = g * L + e
                prods = []
                for u in range(DIM // L):
                    pk = kb[r, pl.ds(u * L, L)]
                    pq = qb[r, pl.ds(u * L, L)]
                    prods.append(pk * pq)
                lane = (iota == e)
                for h in range(HEADS):
                    s = jnp.sum(prods[2 * h] + prods[2 * h + 1])
                    vecs[h] = jnp.where(lane, s, vecs[h])
            for h in range(HEADS):
                scoreb[h, pl.ds(coff + g * L, L)] = vecs[h]
                m = jnp.maximum(m, vecs[h])
            return m

        m = lax.fori_loop(0, BA // L, group_body, m)

        @pl.when(j % FL == FL - 1)
        def _():
            fbase = w * EPW + (j // FL) * (FL * BA)
            for h in range(HEADS):
                pltpu.sync_copy(scoreb.at[h],
                                score_hbm.at[h, pl.ds(fbase, FL * BA)])

        return m

    def pair_body(jo, m):
        m = block_step(2 * jo, m, krows0, qrows0, semk0, semq0,
                       krows1, qrows1, semk1, semq1)
        m = block_step(2 * jo + 1, m, krows1, qrows1, semk1, semq1,
                       krows0, qrows0, semk0, semq0)
        return m

    m = lax.fori_loop(0, NBLKA // 2, pair_body,
                      jnp.full((L,), -3e38, jnp.float32))
    wbuf[...] = m
    pltpu.sync_copy(wbuf, wmax_hbm.at[w])


_score_call = functools.partial(
    pl.kernel,
    out_type=[jax.ShapeDtypeStruct((HEADS, EPAD), jnp.float32),
              jax.ShapeDtypeStruct((NW, L), jnp.float32)],
    mesh=_mesh,
    compiler_params=_sc_params,
    scratch_types=[
        pltpu.VMEM((NBLKA, BA), jnp.int32),
        pltpu.VMEM((NBLKA, BA), jnp.int32),
        pltpu.VMEM((BA, DIM), jnp.float32),
        pltpu.VMEM((BA, DIM), jnp.float32),
        pltpu.VMEM((BA, DIM), jnp.float32),
        pltpu.VMEM((BA, DIM), jnp.float32),
        pltpu.VMEM((HEADS, FL * BA), jnp.float32),
        pltpu.VMEM((L,), jnp.float32),
        pltpu.SemaphoreType.DMA,
        pltpu.SemaphoreType.DMA,
        pltpu.SemaphoreType.DMA,
        pltpu.SemaphoreType.DMA,
    ],
)(_score_body)


def _make_agg_body(hg):
    def _agg_body(score_hbm, wmax_hbm, v2_hbm, src_hbm, dsts_hbm,
                  zacc_hbm, zden_hbm,
                  out_hbm,
                  src_v, dsts_v, ex_hm, exb, vrows0, vrows1, msgb, msgb1,
                  obuf, wmaxv, denb, acc_sp, den_sp,
                  semv0, semv1, semm0, semm1):
        c = lax.axis_index("c")
        t = lax.axis_index("s")
        qq = hg * NC + c

        pltpu.sync_copy(wmax_hbm, wmaxv)
        m = wmaxv[0, :]
        for i in range(1, NW):
            m = jnp.maximum(m, wmaxv[i, :])
        gmax = jnp.max(m)

        rb = t * RPT
        pltpu.sync_copy(zacc_hbm.at[pl.ds(rb, RPT)], acc_sp.at[pl.ds(rb, RPT)])
        pltpu.sync_copy(zden_hbm.at[pl.ds(rb, RPT)], den_sp.at[pl.ds(rb, RPT)])

        pltpu.sync_copy(src_hbm.at[pl.ds(t * BLKB, BLKB)], src_v)
        pltpu.sync_copy(dsts_hbm.at[pl.ds(t * BLKB, BLKB)], dsts_v)

        iota = lax.iota(jnp.int32, L)
        ebase = t * EPT

        for g in range(B // L):
            rows = iota + g * L
            for h in range(DEN_W):
                plsc.store_scatter(exb, [rows, jnp.full((L,), h, jnp.int32)],
                                   jnp.zeros((L,), jnp.float32))

        for h in range(HQ):
            pltpu.sync_copy(score_hbm.at[qq * HQ + h, pl.ds(ebase, EPT)],
                            ex_hm.at[h])

            def ex_body(i, _, h=h):
                sv = ex_hm[h, pl.ds(i * L, L)]
                ev = jnp.exp(sv - gmax)
                gid = iota + (ebase + i * L)
                ev = jnp.where(gid < E, ev, 0.0)
                ex_hm[h, pl.ds(i * L, L)] = ev
                return 0

            lax.fori_loop(0, EPT // L, ex_body, 0, unroll=8)

        plsc.subcore_barrier()

        def den_body(j, _):
            def deng_body(g, _):
                rows = iota + g * L
                for h in range(HQ):
                    ev = ex_hm[h, pl.ds(j * B + g * L, L)]
                    plsc.store_scatter(
                        exb, [rows, jnp.full((L,), h, jnp.int32)], ev)
                return 0

            lax.fori_loop(0, B // L, deng_body, 0, unroll=4)
            pltpu.sync_copy(exb, den_sp.at[dsts_v.at[j]], add=True)
            return 0

        lax.fori_loop(0, BLKB, den_body, 0)

        pltpu.async_copy(v2_hbm.at[qq].at[src_v.at[0]], vrows0, semv0)

        def blk_step(j, vb, vs, nvb, nvs, mb, ms):
            @pl.when(j + 1 < BLKB)
            def _():
                pltpu.async_copy(v2_hbm.at[qq].at[src_v.at[j + 1]], nvb, nvs)

            pltpu.make_async_copy(v2_hbm.at[qq].at[src_v.at[j]], vb, vs).wait()

            @pl.when(j >= 2)
            def _():
                pltpu.make_async_copy(mb, acc_sp.at[dsts_v.at[j - 2]],
                                      ms).wait()

            def g2_body(g, _):
                exvs = [ex_hm[h, pl.ds(j * B + g * L, L)] for h in range(HQ)]
                for e in range(L):
                    r = g * L + e
                    for h in range(HQ):
                        sc = jnp.broadcast_to(exvs[h][e], (L,))
                        for u in range(DK // L):
                            co = h * DK + u * L
                            mb[r, pl.ds(co, L)] = vb[r, pl.ds(co, L)] * sc
                return 0

            lax.fori_loop(0, B // L, g2_body, 0)
            pltpu.make_async_copy(mb, acc_sp.at[dsts_v.at[j]],
                                  ms).start(add=True)

        def pair2_body(jo, _):
            blk_step(2 * jo, vrows0, semv0, vrows1, semv1, msgb, semm0)
            blk_step(2 * jo + 1, vrows1, semv1, vrows0, semv0, msgb1, semm1)
            return 0

        lax.fori_loop(0, BLKB // 2, pair2_body, 0)
        pltpu.make_async_copy(msgb, acc_sp.at[dsts_v.at[BLKB - 2]],
                              semm0).wait()
        pltpu.make_async_copy(msgb1, acc_sp.at[dsts_v.at[BLKB - 1]],
                              semm1).wait()

        plsc.subcore_barrier()

        for off, sz in ((0, 128), (128, 128), (256, 128), (384, 128), (512, 113)):
            rs = rb + off
            pltpu.sync_copy(acc_sp.at[pl.ds(rs, sz)], vrows0.at[pl.ds(0, sz)])
            pltpu.sync_copy(den_sp.at[pl.ds(rs, sz)], denb.at[pl.ds(0, sz)])
            def g3_body(g, _):
                rows = iota + g * L
                rvs = []
                for h in range(HQ):
                    dv = plsc.load_gather(
                        denb, [rows, jnp.full((L,), h, jnp.int32)])
                    rvs.append(1.0 / (dv + 1e-9))
                for e in range(L):
                    r = g * L + e
                    for h in range(HQ):
                        sc = jnp.broadcast_to(rvs[h][e], (L,))
                        for u in range(DK // L):
                            co = h * DK + u * L
                            obuf[r, pl.ds(co, L)] = vrows0[r, pl.ds(co, L)] * sc
                return 0

            lax.fori_loop(0, B // L, g3_body, 0)
            pltpu.sync_copy(obuf.at[pl.ds(0, sz)], out_hbm.at[c, pl.ds(rs, sz)])

    return _agg_body


def _make_agg_call(hg):
    return functools.partial(
        pl.kernel,
        out_type=jax.ShapeDtypeStruct((NC, N, CQ), jnp.float32),
        mesh=_mesh,
        compiler_params=_sc_params,
        scratch_types=[
            pltpu.VMEM((BLKB, B), jnp.int32),
            pltpu.VMEM((BLKB, B), jnp.int32),
            pltpu.VMEM((HQ, EPT), jnp.float32),
            pltpu.VMEM((B, DEN_W), jnp.float32),
            pltpu.VMEM((B, CQ), jnp.float32),
            pltpu.VMEM((B, CQ), jnp.float32),
            pltpu.VMEM((B, CQ), jnp.float32),
            pltpu.VMEM((B, CQ), jnp.float32),
            pltpu.VMEM((B, CQ), jnp.float32),
            pltpu.VMEM((NW, L), jnp.float32),
            pltpu.VMEM((B, DEN_W), jnp.float32),
            pltpu.VMEM_SHARED((N, CQ), jnp.float32),
            pltpu.VMEM_SHARED((N, DEN_W), jnp.float32),
            pltpu.SemaphoreType.DMA,
            pltpu.SemaphoreType.DMA,
            pltpu.SemaphoreType.DMA,
            pltpu.SemaphoreType.DMA,
        ],
    )(_make_agg_body(hg))


_agg_call_0 = _make_agg_call(0)
_agg_call_1 = _make_agg_call(1)


def _proj_ln_body(agg_ref, hd_ref, pw_ref, pb_ref, g_ref, b_ref, y_ref):
    dn = (((1,), (1,)), ((), ()))
    out = pb_ref[...][None, :]
    for qq in range(4):
        out = out + lax.dot_general(agg_ref[qq], pw_ref[:, qq * CQ:(qq + 1) * CQ],
                                    dn, preferred_element_type=jnp.float32)
    res = hd_ref[...] + out
    mu = jnp.mean(res, axis=-1, keepdims=True)
    var = jnp.mean((res - mu) * (res - mu), axis=-1, keepdims=True)
    y_ref[...] = (res - mu) * lax.rsqrt(var + 1e-5) * g_ref[...][None, :] + b_ref[...][None, :]


def _proj_ln(agg4, h_dst, proj_w, proj_b, ln_g, ln_b):
    grid = (N // ROW_BLK,)
    row_spec = pl.BlockSpec((ROW_BLK, DIM), lambda i: (i, 0))
    agg_spec = pl.BlockSpec((4, ROW_BLK, CQ), lambda i: (0, i, 0))
    w_spec = pl.BlockSpec((DIM, DIM), lambda i: (0, 0))
    b_spec = pl.BlockSpec((DIM,), lambda i: (0,))
    return pl.pallas_call(
        _proj_ln_body,
        grid=grid,
        in_specs=[agg_spec, row_spec, w_spec, b_spec, b_spec, b_spec],
        out_specs=row_spec,
        out_shape=jax.ShapeDtypeStruct((N, DIM), jnp.float32),
    )(agg4, h_dst, proj_w, proj_b, ln_g, ln_b)


def kernel(h_src, h_dst, edge_index, Wq_w, Wq_b, Wk_w, Wk_b, Wv_w, Wv_b,
           proj_w, proj_b, ln_g, ln_b):
    src = edge_index[0]
    dst = edge_index[1]
    padz = jnp.zeros((EPAD - E,), jnp.int32)
    src_f = jnp.concatenate([src, padz])
    dst_f = jnp.concatenate([dst, padz])
    src_p = src_f.reshape(EPAD // B, B)
    dst_p = dst_f.reshape(EPAD // B, B)
    src_a = src_f.reshape(EPAD // BA, BA)
    dst_a = dst_f.reshape(EPAD // BA, BA)

    q, k, v2 = _qkv(h_dst, h_src, Wq_w, Wk_w, Wv_w, Wq_b, Wk_b, Wv_b)

    score, wmax = _score_call(k, q, src_a, dst_a)

    zacc = jnp.zeros((N, CQ), jnp.float32)
    zden = jnp.zeros((N, DEN_W), jnp.float32)
    agg_a = _agg_call_0(score, wmax, v2, src_p, dst_p, zacc, zden)
    agg_b = _agg_call_1(score, wmax, v2, src_p, dst_p, zacc, zden)
    agg4 = jnp.concatenate([agg_a, agg_b], axis=0)

    return _proj_ln(agg4, h_dst, proj_w, proj_b, ln_g, ln_b)
